# baseline (speedup 1.0000x reference)
import functools

import jax
import jax.numpy as jnp
from jax import lax
from jax.experimental import pallas as pl
from jax.experimental.pallas import tpu as pltpu
from jax.experimental.pallas import tpu_sc as plsc

BATCH = 1024
D_MODEL = 64

try:
    _info = plsc.get_sparse_core_info()
    _NC, _NS = _info.num_cores, _info.num_subcores
except Exception:
    _NC, _NS = 2, 16
_NW = _NC * _NS
_B_PER_W = BATCH // _NW


def _gather_sc(table, idx):
    mesh = plsc.VectorSubcoreMesh(core_axis_name="c", subcore_axis_name="s")

    @functools.partial(
        pl.kernel,
        mesh=mesh,
        out_type=jax.ShapeDtypeStruct((BATCH, D_MODEL), jnp.float32),
        scratch_types=[
            pltpu.VMEM((_B_PER_W,), jnp.int32),
            pltpu.VMEM((_B_PER_W, D_MODEL), jnp.float32),
            pltpu.SemaphoreType.DMA,
        ],
        compiler_params=pltpu.CompilerParams(use_tc_tiling_on_sc=False),
    )
    def k(table_hbm, idx_hbm, out_hbm, idx_v, rows_v, sem):
        wid = lax.axis_index("s") * _NC + lax.axis_index("c")
        base = wid * _B_PER_W
        pltpu.sync_copy(idx_hbm.at[pl.ds(base, _B_PER_W)], idx_v)
        pltpu.async_copy(table_hbm.at[idx_v], rows_v, sem).wait()
        pltpu.sync_copy(rows_v, out_hbm.at[pl.ds(base, _B_PER_W)])

    return k(table, idx)


_VBLK = 2048


def _mm_body(emb_ref, w_ref, b_ref, out_ref):
    acc = lax.dot_general(
        emb_ref[...],
        w_ref[...],
        (((1,), (1,)), ((), ())),
        preferred_element_type=jnp.float32,
    )
    out_ref[...] = acc + b_ref[...]


def _project_tc(emb, W, b2):
    vocab = W.shape[0]
    grid = (pl.cdiv(vocab, _VBLK),)
    return pl.pallas_call(
        _mm_body,
        grid=grid,
        in_specs=[
            pl.BlockSpec((BATCH, D_MODEL), lambda j: (0, 0)),
            pl.BlockSpec((_VBLK, D_MODEL), lambda j: (j, 0)),
            pl.BlockSpec((1, _VBLK), lambda j: (0, j)),
        ],
        out_specs=pl.BlockSpec((BATCH, _VBLK), lambda j: (0, j)),
        out_shape=jax.ShapeDtypeStruct((BATCH, vocab), jnp.float32),
    )(emb, W, b2)


def kernel(x, table, W, b):
    emb = _gather_sc(table, x.astype(jnp.int32))
    return _project_tc(emb, W, b.reshape(1, -1))


# speedup vs baseline: 2.2430x; 2.2430x over previous
import functools

import jax
import jax.numpy as jnp
from jax import lax
from jax.experimental import pallas as pl
from jax.experimental.pallas import tpu as pltpu
from jax.experimental.pallas import tpu_sc as plsc

BATCH = 1024
D_MODEL = 64

try:
    _info = plsc.get_sparse_core_info()
    _NC, _NS = _info.num_cores, _info.num_subcores
except Exception:
    _NC, _NS = 2, 16
_NW = _NC * _NS
_B_PER_W = BATCH // _NW


def _gather_sc(table, idx):
    mesh = plsc.VectorSubcoreMesh(core_axis_name="c", subcore_axis_name="s")

    @functools.partial(
        pl.kernel,
        mesh=mesh,
        out_type=jax.ShapeDtypeStruct((BATCH, D_MODEL), jnp.float32),
        scratch_types=[
            pltpu.VMEM((_B_PER_W,), jnp.int32),
            pltpu.VMEM((_B_PER_W, D_MODEL), jnp.float32),
            pltpu.SemaphoreType.DMA,
        ],
        compiler_params=pltpu.CompilerParams(use_tc_tiling_on_sc=False),
    )
    def k(table_hbm, idx_hbm, out_hbm, idx_v, rows_v, sem):
        wid = lax.axis_index("s") * _NC + lax.axis_index("c")
        base = wid * _B_PER_W
        pltpu.sync_copy(idx_hbm.at[pl.ds(base, _B_PER_W)], idx_v)
        pltpu.async_copy(table_hbm.at[idx_v], rows_v, sem).wait()
        pltpu.sync_copy(rows_v, out_hbm.at[pl.ds(base, _B_PER_W)])

    return k(table, idx)


_VBLK = 2048


def _mmT_body(wt_ref, emb_ref, b_ref, out_ref):
    acc = lax.dot_general(
        wt_ref[...],
        emb_ref[...],
        (((0,), (1,)), ((), ())),
        preferred_element_type=jnp.float32,
    )
    out_ref[...] = acc + b_ref[...]


def _project_tc_T(emb, WT, b1):
    vocab = WT.shape[1]
    grid = (pl.cdiv(vocab, _VBLK),)
    return pl.pallas_call(
        _mmT_body,
        grid=grid,
        in_specs=[
            pl.BlockSpec((D_MODEL, _VBLK), lambda j: (0, j)),
            pl.BlockSpec((BATCH, D_MODEL), lambda j: (0, 0)),
            pl.BlockSpec((_VBLK, 1), lambda j: (j, 0)),
        ],
        out_specs=pl.BlockSpec((_VBLK, BATCH), lambda j: (j, 0)),
        out_shape=jax.ShapeDtypeStruct((vocab, BATCH), jnp.float32),
    )(WT, emb, b1)


def kernel(x, table, W, b):
    emb = _gather_sc(table, x.astype(jnp.int32))
    logits_t = _project_tc_T(emb, W.T, b.reshape(-1, 1))
    return logits_t.T


# speedup vs baseline: 2.8242x; 1.2591x over previous
import functools

import jax
import jax.numpy as jnp
from jax import lax
from jax.experimental import pallas as pl
from jax.experimental.pallas import tpu as pltpu
from jax.experimental.pallas import tpu_sc as plsc

BATCH = 1024
D_MODEL = 64

try:
    _info = plsc.get_sparse_core_info()
    _NC, _NS = _info.num_cores, _info.num_subcores
except Exception:
    _NC, _NS = 2, 16
_NW = _NC * _NS
_B_PER_W = BATCH // _NW
_L = 16


def _gather_sc(table2, idx):
    mesh = plsc.VectorSubcoreMesh(core_axis_name="c", subcore_axis_name="s")

    @functools.partial(
        pl.kernel,
        mesh=mesh,
        out_type=jax.ShapeDtypeStruct((BATCH, D_MODEL), jnp.float32),
        scratch_types=[
            pltpu.VMEM((_B_PER_W,), jnp.int32),
            pltpu.VMEM((_B_PER_W,), jnp.int32),
            pltpu.VMEM((_B_PER_W, 2 * D_MODEL), jnp.float32),
            pltpu.VMEM((_B_PER_W, D_MODEL), jnp.float32),
            pltpu.SemaphoreType.DMA,
        ],
    )
    def k(table_hbm, idx_hbm, out_hbm, idx_v, idx2_v, pairs_v, sel_v, sem):
        wid = lax.axis_index("s") * _NC + lax.axis_index("c")
        base = wid * _B_PER_W
        pltpu.sync_copy(idx_hbm.at[pl.ds(base, _B_PER_W)], idx_v)
        for c in range(_B_PER_W // _L):
            chunk = idx_v[pl.ds(c * _L, _L)]
            idx2_v[pl.ds(c * _L, _L)] = lax.shift_right_logical(chunk, 1)
        pltpu.async_copy(table_hbm.at[idx2_v], pairs_v, sem).wait()
        for c in range(_B_PER_W // _L):
            par = idx_v[pl.ds(c * _L, _L)] & 1
            for r in range(_L):
                row = c * _L + r
                par_r = jnp.take(par, jnp.full((_L,), r, jnp.int32))
                parf = par_r.astype(jnp.float32)
                for k4 in range(D_MODEL // _L):
                    lo = pairs_v[row, pl.ds(k4 * _L, _L)]
                    hi = pairs_v[row, pl.ds(D_MODEL + k4 * _L, _L)]
                    sel_v[row, pl.ds(k4 * _L, _L)] = lo + parf * (hi - lo)
        pltpu.sync_copy(sel_v, out_hbm.at[pl.ds(base, _B_PER_W)])

    return k(table2, idx)


_VBLK = 2048


def _mmT_body(wt_ref, emb_ref, b_ref, out_ref):
    acc = lax.dot_general(
        wt_ref[...],
        emb_ref[...],
        (((0,), (1,)), ((), ())),
        preferred_element_type=jnp.float32,
    )
    out_ref[...] = acc + jnp.swapaxes(b_ref[...], 0, 1)


def _project_tc_T(emb, WT, b2):
    vocab = WT.shape[1]
    grid = (pl.cdiv(vocab, _VBLK),)
    return pl.pallas_call(
        _mmT_body,
        grid=grid,
        in_specs=[
            pl.BlockSpec((D_MODEL, _VBLK), lambda j: (0, j)),
            pl.BlockSpec((BATCH, D_MODEL), lambda j: (0, 0)),
            pl.BlockSpec((1, _VBLK), lambda j: (0, j)),
        ],
        out_specs=pl.BlockSpec((_VBLK, BATCH), lambda j: (j, 0)),
        out_shape=jax.ShapeDtypeStruct((vocab, BATCH), jnp.float32),
    )(WT, emb, b2)


def kernel(x, table, W, b):
    table2 = table.reshape(-1, 2 * D_MODEL)
    emb = _gather_sc(table2, x.astype(jnp.int32))
    logits_t = _project_tc_T(emb, W.T, b.reshape(1, -1))
    return logits_t.T


# speedup vs baseline: 2.8657x; 1.0147x over previous
import functools

import jax
import jax.numpy as jnp
from jax import lax
from jax.experimental import pallas as pl
from jax.experimental.pallas import tpu as pltpu
from jax.experimental.pallas import tpu_sc as plsc

BATCH = 1024
D_MODEL = 64

try:
    _info = plsc.get_sparse_core_info()
    _NC, _NS = _info.num_cores, _info.num_subcores
except Exception:
    _NC, _NS = 2, 16
_NW = _NC * _NS
_B_PER_W = BATCH // _NW
_L = 16


def _gather_sc(table2, idx):
    mesh = plsc.VectorSubcoreMesh(core_axis_name="c", subcore_axis_name="s")

    @functools.partial(
        pl.kernel,
        mesh=mesh,
        out_type=jax.ShapeDtypeStruct((BATCH, D_MODEL), jnp.float32),
        scratch_types=[
            pltpu.VMEM((_B_PER_W,), jnp.int32),
            pltpu.VMEM((_B_PER_W,), jnp.int32),
            pltpu.VMEM((_B_PER_W, 2 * D_MODEL), jnp.float32),
            pltpu.VMEM((_B_PER_W, D_MODEL), jnp.float32),
            pltpu.SemaphoreType.DMA,
        ],
    )
    def k(table_hbm, idx_hbm, out_hbm, idx_v, idx2_v, pairs_v, sel_v, sem):
        wid = lax.axis_index("s") * _NC + lax.axis_index("c")
        base = wid * _B_PER_W
        pltpu.sync_copy(idx_hbm.at[pl.ds(base, _B_PER_W)], idx_v)
        for c in range(_B_PER_W // _L):
            chunk = idx_v[pl.ds(c * _L, _L)]
            idx2_v[pl.ds(c * _L, _L)] = lax.shift_right_logical(chunk, 1)
        pltpu.async_copy(table_hbm.at[idx2_v], pairs_v, sem).wait()
        for c in range(_B_PER_W // _L):
            par = idx_v[pl.ds(c * _L, _L)] & 1
            for r in range(_L):
                row = c * _L + r
                par_r = jnp.take(par, jnp.full((_L,), r, jnp.int32))
                parf = par_r.astype(jnp.float32)
                for k4 in range(D_MODEL // _L):
                    lo = pairs_v[row, pl.ds(k4 * _L, _L)]
                    hi = pairs_v[row, pl.ds(D_MODEL + k4 * _L, _L)]
                    sel_v[row, pl.ds(k4 * _L, _L)] = lo + parf * (hi - lo)
        pltpu.sync_copy(sel_v, out_hbm.at[pl.ds(base, _B_PER_W)])

    return k(table2, idx)


_VBLK = 4096


def _mmT_body(wt_ref, emb_ref, b_ref, out_ref):
    acc = lax.dot_general(
        wt_ref[...],
        emb_ref[...],
        (((0,), (1,)), ((), ())),
        preferred_element_type=jnp.float32,
    )
    out_ref[...] = acc + jnp.swapaxes(b_ref[...], 0, 1)


def _project_tc_T(emb, WT, b2):
    vocab = WT.shape[1]
    grid = (pl.cdiv(vocab, _VBLK),)
    return pl.pallas_call(
        _mmT_body,
        grid=grid,
        in_specs=[
            pl.BlockSpec((D_MODEL, _VBLK), lambda j: (0, j)),
            pl.BlockSpec((BATCH, D_MODEL), lambda j: (0, 0)),
            pl.BlockSpec((1, _VBLK), lambda j: (0, j)),
        ],
        out_specs=pl.BlockSpec((_VBLK, BATCH), lambda j: (j, 0)),
        out_shape=jax.ShapeDtypeStruct((vocab, BATCH), jnp.float32),
    )(WT, emb, b2)


def kernel(x, table, W, b):
    table2 = table.reshape(-1, 2 * D_MODEL)
    emb = _gather_sc(table2, x.astype(jnp.int32))
    logits_t = _project_tc_T(emb, W.T, b.reshape(1, -1))
    return logits_t.T


# speedup vs baseline: 3.1361x; 1.0944x over previous
import functools

import jax
import jax.numpy as jnp
from jax import lax
from jax.experimental import pallas as pl
from jax.experimental.pallas import tpu as pltpu
from jax.experimental.pallas import tpu_sc as plsc

BATCH = 1024
D_MODEL = 64

try:
    _info = plsc.get_sparse_core_info()
    _NC, _NS = _info.num_cores, _info.num_subcores
except Exception:
    _NC, _NS = 2, 16
_NW = _NC * _NS
_B_PER_W = BATCH // _NW
_L = 16


_PBLK = 4096


def _prep_body(tt_ref, out_ref):
    x = tt_ref[...]
    xt = jnp.swapaxes(x, 0, 1)
    x3 = xt.reshape(_PBLK // 2, 2, D_MODEL)
    out_ref[...] = jnp.concatenate([x3[:, 0, :], x3[:, 1, :]], axis=1)


def _prep_tc(tableT):
    vocab = tableT.shape[1]
    grid = (pl.cdiv(vocab, _PBLK),)
    return pl.pallas_call(
        _prep_body,
        grid=grid,
        in_specs=[pl.BlockSpec((D_MODEL, _PBLK), lambda j: (0, j))],
        out_specs=pl.BlockSpec((_PBLK // 2, 2 * D_MODEL), lambda j: (j, 0)),
        out_shape=jax.ShapeDtypeStruct((vocab // 2, 2 * D_MODEL), jnp.float32),
    )(tableT)


def _gather_sc(table2, idx):
    mesh = plsc.VectorSubcoreMesh(core_axis_name="c", subcore_axis_name="s")

    @functools.partial(
        pl.kernel,
        mesh=mesh,
        out_type=jax.ShapeDtypeStruct((BATCH, D_MODEL), jnp.float32),
        scratch_types=[
            pltpu.VMEM((_B_PER_W,), jnp.int32),
            pltpu.VMEM((_B_PER_W,), jnp.int32),
            pltpu.VMEM((_B_PER_W, 2 * D_MODEL), jnp.float32),
            pltpu.VMEM((_B_PER_W, D_MODEL), jnp.float32),
            pltpu.SemaphoreType.DMA,
        ],
    )
    def k(table_hbm, idx_hbm, out_hbm, idx_v, idx2_v, pairs_v, sel_v, sem):
        wid = lax.axis_index("s") * _NC + lax.axis_index("c")
        base = wid * _B_PER_W
        pltpu.sync_copy(idx_hbm.at[pl.ds(base, _B_PER_W)], idx_v)
        for c in range(_B_PER_W // _L):
            chunk = idx_v[pl.ds(c * _L, _L)]
            idx2_v[pl.ds(c * _L, _L)] = lax.shift_right_logical(chunk, 1)
        pltpu.async_copy(table_hbm.at[idx2_v], pairs_v, sem).wait()
        for c in range(_B_PER_W // _L):
            par = idx_v[pl.ds(c * _L, _L)] & 1
            for r in range(_L):
                row = c * _L + r
                par_r = jnp.take(par, jnp.full((_L,), r, jnp.int32))
                parf = par_r.astype(jnp.float32)
                for k4 in range(D_MODEL // _L):
                    lo = pairs_v[row, pl.ds(k4 * _L, _L)]
                    hi = pairs_v[row, pl.ds(D_MODEL + k4 * _L, _L)]
                    sel_v[row, pl.ds(k4 * _L, _L)] = lo + parf * (hi - lo)
        pltpu.sync_copy(sel_v, out_hbm.at[pl.ds(base, _B_PER_W)])

    return k(table2, idx)


_VBLK = 4096


def _mmT_body(wt_ref, emb_ref, b_ref, out_ref):
    acc = lax.dot_general(
        wt_ref[...],
        emb_ref[...],
        (((0,), (1,)), ((), ())),
        preferred_element_type=jnp.float32,
    )
    out_ref[...] = acc + jnp.swapaxes(b_ref[...], 0, 1)


def _project_tc_T(emb, WT, b2):
    vocab = WT.shape[1]
    grid = (pl.cdiv(vocab, _VBLK),)
    return pl.pallas_call(
        _mmT_body,
        grid=grid,
        in_specs=[
            pl.BlockSpec((D_MODEL, _VBLK), lambda j: (0, j)),
            pl.BlockSpec((BATCH, D_MODEL), lambda j: (0, 0)),
            pl.BlockSpec((1, _VBLK), lambda j: (0, j)),
        ],
        out_specs=pl.BlockSpec((_VBLK, BATCH), lambda j: (j, 0)),
        out_shape=jax.ShapeDtypeStruct((vocab, BATCH), jnp.float32),
    )(WT, emb, b2)


def kernel(x, table, W, b):
    table2 = _prep_tc(table.T)
    emb = _gather_sc(table2, x.astype(jnp.int32))
    logits_t = _project_tc_T(emb, W.T, b.reshape(1, -1))
    return logits_t.T


# speedup vs baseline: 3.3093x; 1.0552x over previous
import functools

import jax
import jax.numpy as jnp
from jax import lax
from jax.experimental import pallas as pl
from jax.experimental.pallas import tpu as pltpu
from jax.experimental.pallas import tpu_sc as plsc

BATCH = 1024
D_MODEL = 64

try:
    _info = plsc.get_sparse_core_info()
    _NC, _NS = _info.num_cores, _info.num_subcores
except Exception:
    _NC, _NS = 2, 16
_NW = _NC * _NS
_B_PER_W = BATCH // _NW
_L = 16


_PBLK = 4096


_HALF = _PBLK // 2


def _prep_body(tt_ref, out_ref):
    xt = jnp.swapaxes(tt_ref[...], 0, 1)
    out_ref[:, 0:D_MODEL] = xt[0:_HALF]
    out_ref[:, D_MODEL:] = xt[_HALF:]


def _prep_tc(tableT):
    vocab = tableT.shape[1]
    n_chunks = pl.cdiv(vocab, _PBLK)
    return pl.pallas_call(
        _prep_body,
        grid=(n_chunks,),
        in_specs=[pl.BlockSpec((D_MODEL, _PBLK), lambda j: (0, j))],
        out_specs=pl.BlockSpec((_HALF, 2 * D_MODEL), lambda j: (j, 0)),
        out_shape=jax.ShapeDtypeStruct((n_chunks * _HALF, 2 * D_MODEL), jnp.float32),
    )(tableT)


def _gather_sc(table2, idx):
    mesh = plsc.VectorSubcoreMesh(core_axis_name="c", subcore_axis_name="s")

    @functools.partial(
        pl.kernel,
        mesh=mesh,
        out_type=jax.ShapeDtypeStruct((BATCH, D_MODEL), jnp.float32),
        scratch_types=[
            pltpu.VMEM((_B_PER_W,), jnp.int32),
            pltpu.VMEM((_B_PER_W,), jnp.int32),
            pltpu.VMEM((_B_PER_W, 2 * D_MODEL), jnp.float32),
            pltpu.VMEM((_B_PER_W, D_MODEL), jnp.float32),
            pltpu.SemaphoreType.DMA,
        ],
    )
    def k(table_hbm, idx_hbm, out_hbm, idx_v, idx2_v, pairs_v, sel_v, sem):
        wid = lax.axis_index("s") * _NC + lax.axis_index("c")
        base = wid * _B_PER_W
        pltpu.sync_copy(idx_hbm.at[pl.ds(base, _B_PER_W)], idx_v)
        for c in range(_B_PER_W // _L):
            chunk = idx_v[pl.ds(c * _L, _L)]
            row = lax.shift_left(lax.shift_right_logical(chunk, 12), 11) + (
                chunk & (_HALF - 1))
            idx2_v[pl.ds(c * _L, _L)] = row
        pltpu.async_copy(table_hbm.at[idx2_v], pairs_v, sem).wait()
        for c in range(_B_PER_W // _L):
            par = lax.shift_right_logical(idx_v[pl.ds(c * _L, _L)], 11) & 1
            for r in range(_L):
                row = c * _L + r
                par_r = jnp.take(par, jnp.full((_L,), r, jnp.int32))
                parf = par_r.astype(jnp.float32)
                for k4 in range(D_MODEL // _L):
                    lo = pairs_v[row, pl.ds(k4 * _L, _L)]
                    hi = pairs_v[row, pl.ds(D_MODEL + k4 * _L, _L)]
                    sel_v[row, pl.ds(k4 * _L, _L)] = lo + parf * (hi - lo)
        pltpu.sync_copy(sel_v, out_hbm.at[pl.ds(base, _B_PER_W)])

    return k(table2, idx)


_VBLK = 4096


def _mmT_body(wt_ref, emb_ref, b_ref, out_ref):
    acc = lax.dot_general(
        wt_ref[...],
        emb_ref[...],
        (((0,), (1,)), ((), ())),
        preferred_element_type=jnp.float32,
    )
    out_ref[...] = acc + jnp.swapaxes(b_ref[...], 0, 1)


def _project_tc_T(emb, WT, b2):
    vocab = WT.shape[1]
    grid = (pl.cdiv(vocab, _VBLK),)
    return pl.pallas_call(
        _mmT_body,
        grid=grid,
        in_specs=[
            pl.BlockSpec((D_MODEL, _VBLK), lambda j: (0, j)),
            pl.BlockSpec((BATCH, D_MODEL), lambda j: (0, 0)),
            pl.BlockSpec((1, _VBLK), lambda j: (0, j)),
        ],
        out_specs=pl.BlockSpec((_VBLK, BATCH), lambda j: (j, 0)),
        out_shape=jax.ShapeDtypeStruct((vocab, BATCH), jnp.float32),
    )(WT, emb, b2)


def kernel(x, table, W, b):
    table2 = _prep_tc(table.T)
    emb = _gather_sc(table2, x.astype(jnp.int32))
    logits_t = _project_tc_T(emb, W.T, b.reshape(1, -1))
    return logits_t.T


# speedup vs baseline: 3.3950x; 1.0259x over previous
import functools

import jax
import jax.numpy as jnp
from jax import lax
from jax.experimental import pallas as pl
from jax.experimental.pallas import tpu as pltpu
from jax.experimental.pallas import tpu_sc as plsc

BATCH = 1024
D_MODEL = 64

try:
    _info = plsc.get_sparse_core_info()
    _NC, _NS = _info.num_cores, _info.num_subcores
except Exception:
    _NC, _NS = 2, 16
_NW = _NC * _NS
_B_PER_W = BATCH // _NW
_L = 16


_PBLK = 8192


_HALF = _PBLK // 2


def _prep_body(tt_ref, eye_ref, out_ref):
    xt = lax.dot_general(
        tt_ref[...], eye_ref[...],
        (((0,), (0,)), ((), ())),
        preferred_element_type=jnp.float32,
    )
    out_ref[:, 0:D_MODEL] = xt[0:_HALF]
    out_ref[:, D_MODEL:] = xt[_HALF:]


def _prep_tc(tableT):
    vocab = tableT.shape[1]
    n_chunks = pl.cdiv(vocab, _PBLK)
    return pl.pallas_call(
        _prep_body,
        grid=(n_chunks,),
        in_specs=[
            pl.BlockSpec((D_MODEL, _PBLK), lambda j: (0, j)),
            pl.BlockSpec((D_MODEL, D_MODEL), lambda j: (0, 0)),
        ],
        out_specs=pl.BlockSpec((_HALF, 2 * D_MODEL), lambda j: (j, 0)),
        out_shape=jax.ShapeDtypeStruct((n_chunks * _HALF, 2 * D_MODEL), jnp.float32),
    )(tableT, jnp.eye(D_MODEL, dtype=jnp.float32))


def _gather_sc(table2, idx):
    mesh = plsc.VectorSubcoreMesh(core_axis_name="c", subcore_axis_name="s")

    @functools.partial(
        pl.kernel,
        mesh=mesh,
        out_type=jax.ShapeDtypeStruct((BATCH, D_MODEL), jnp.float32),
        scratch_types=[
            pltpu.VMEM((_B_PER_W,), jnp.int32),
            pltpu.VMEM((_B_PER_W,), jnp.int32),
            pltpu.VMEM((_B_PER_W, 2 * D_MODEL), jnp.float32),
            pltpu.VMEM((_B_PER_W, D_MODEL), jnp.float32),
            pltpu.SemaphoreType.DMA,
        ],
    )
    def k(table_hbm, idx_hbm, out_hbm, idx_v, idx2_v, pairs_v, sel_v, sem):
        wid = lax.axis_index("s") * _NC + lax.axis_index("c")
        base = wid * _B_PER_W
        pltpu.sync_copy(idx_hbm.at[pl.ds(base, _B_PER_W)], idx_v)
        for c in range(_B_PER_W // _L):
            chunk = idx_v[pl.ds(c * _L, _L)]
            row = lax.shift_left(lax.shift_right_logical(chunk, 13), 12) + (
                chunk & (_HALF - 1))
            idx2_v[pl.ds(c * _L, _L)] = row
        pltpu.async_copy(table_hbm.at[idx2_v], pairs_v, sem).wait()
        for c in range(_B_PER_W // _L):
            par = lax.shift_right_logical(idx_v[pl.ds(c * _L, _L)], 12) & 1
            for r in range(_L):
                row = c * _L + r
                par_r = jnp.take(par, jnp.full((_L,), r, jnp.int32))
                parf = par_r.astype(jnp.float32)
                for k4 in range(D_MODEL // _L):
                    lo = pairs_v[row, pl.ds(k4 * _L, _L)]
                    hi = pairs_v[row, pl.ds(D_MODEL + k4 * _L, _L)]
                    sel_v[row, pl.ds(k4 * _L, _L)] = lo + parf * (hi - lo)
        pltpu.sync_copy(sel_v, out_hbm.at[pl.ds(base, _B_PER_W)])

    return k(table2, idx)


_VBLK = 4096


def _mmT_body(wt_ref, emb_ref, b_ref, out_ref):
    acc = lax.dot_general(
        wt_ref[...],
        emb_ref[...],
        (((0,), (1,)), ((), ())),
        preferred_element_type=jnp.float32,
    )
    out_ref[...] = acc + jnp.swapaxes(b_ref[...], 0, 1)


def _project_tc_T(emb, WT, b2):
    vocab = WT.shape[1]
    grid = (pl.cdiv(vocab, _VBLK),)
    return pl.pallas_call(
        _mmT_body,
        grid=grid,
        in_specs=[
            pl.BlockSpec((D_MODEL, _VBLK), lambda j: (0, j)),
            pl.BlockSpec((BATCH, D_MODEL), lambda j: (0, 0)),
            pl.BlockSpec((1, _VBLK), lambda j: (0, j)),
        ],
        out_specs=pl.BlockSpec((_VBLK, BATCH), lambda j: (j, 0)),
        out_shape=jax.ShapeDtypeStruct((vocab, BATCH), jnp.float32),
    )(WT, emb, b2)


def kernel(x, table, W, b):
    table2 = _prep_tc(table.T)
    emb = _gather_sc(table2, x.astype(jnp.int32))
    logits_t = _project_tc_T(emb, W.T, b.reshape(1, -1))
    return logits_t.T


# speedup vs baseline: 3.4020x; 1.0021x over previous
import functools

import jax
import jax.numpy as jnp
from jax import lax
from jax.experimental import pallas as pl
from jax.experimental.pallas import tpu as pltpu
from jax.experimental.pallas import tpu_sc as plsc

BATCH = 1024
D_MODEL = 64

try:
    _info = plsc.get_sparse_core_info()
    _NC, _NS = _info.num_cores, _info.num_subcores
except Exception:
    _NC, _NS = 2, 16
_NW = _NC * _NS
_B_PER_W = BATCH // _NW
_L = 16


_PBLK = 16384


_HALF = _PBLK // 2


def _prep_body(tt_ref, eye_ref, out_ref):
    xt = lax.dot_general(
        tt_ref[...], eye_ref[...],
        (((0,), (0,)), ((), ())),
        preferred_element_type=jnp.float32,
    )
    out_ref[:, 0:D_MODEL] = xt[0:_HALF]
    out_ref[:, D_MODEL:] = xt[_HALF:]


def _prep_tc(tableT):
    vocab = tableT.shape[1]
    n_chunks = pl.cdiv(vocab, _PBLK)
    return pl.pallas_call(
        _prep_body,
        grid=(n_chunks,),
        in_specs=[
            pl.BlockSpec((D_MODEL, _PBLK), lambda j: (0, j)),
            pl.BlockSpec((D_MODEL, D_MODEL), lambda j: (0, 0)),
        ],
        out_specs=pl.BlockSpec((_HALF, 2 * D_MODEL), lambda j: (j, 0)),
        out_shape=jax.ShapeDtypeStruct((n_chunks * _HALF, 2 * D_MODEL), jnp.float32),
    )(tableT, jnp.eye(D_MODEL, dtype=jnp.float32))


def _gather_sc(table2, idx):
    mesh = plsc.VectorSubcoreMesh(core_axis_name="c", subcore_axis_name="s")

    @functools.partial(
        pl.kernel,
        mesh=mesh,
        out_type=jax.ShapeDtypeStruct((BATCH, D_MODEL), jnp.float32),
        scratch_types=[
            pltpu.VMEM((_B_PER_W,), jnp.int32),
            pltpu.VMEM((_B_PER_W,), jnp.int32),
            pltpu.VMEM((_B_PER_W, 2 * D_MODEL), jnp.float32),
            pltpu.VMEM((_B_PER_W, D_MODEL), jnp.float32),
            pltpu.SemaphoreType.DMA,
        ],
    )
    def k(table_hbm, idx_hbm, out_hbm, idx_v, idx2_v, pairs_v, sel_v, sem):
        wid = lax.axis_index("s") * _NC + lax.axis_index("c")
        base = wid * _B_PER_W
        pltpu.sync_copy(idx_hbm.at[pl.ds(base, _B_PER_W)], idx_v)
        for c in range(_B_PER_W // _L):
            chunk = idx_v[pl.ds(c * _L, _L)]
            row = lax.shift_left(lax.shift_right_logical(chunk, 14), 13) + (
                chunk & (_HALF - 1))
            idx2_v[pl.ds(c * _L, _L)] = row
        pltpu.async_copy(table_hbm.at[idx2_v], pairs_v, sem).wait()
        for c in range(_B_PER_W // _L):
            par = lax.shift_right_logical(idx_v[pl.ds(c * _L, _L)], 13) & 1
            for r in range(_L):
                row = c * _L + r
                par_r = jnp.take(par, jnp.full((_L,), r, jnp.int32))
                parf = par_r.astype(jnp.float32)
                for k4 in range(D_MODEL // _L):
                    lo = pairs_v[row, pl.ds(k4 * _L, _L)]
                    hi = pairs_v[row, pl.ds(D_MODEL + k4 * _L, _L)]
                    sel_v[row, pl.ds(k4 * _L, _L)] = lo + parf * (hi - lo)
        pltpu.sync_copy(sel_v, out_hbm.at[pl.ds(base, _B_PER_W)])

    return k(table2, idx)


_VBLK = 4096


def _mmT_body(wt_ref, emb_ref, b_ref, out_ref):
    acc = lax.dot_general(
        wt_ref[...],
        emb_ref[...],
        (((0,), (1,)), ((), ())),
        preferred_element_type=jnp.float32,
    )
    out_ref[...] = acc + jnp.swapaxes(b_ref[...], 0, 1)


def _project_tc_T(emb, WT, b2):
    vocab = WT.shape[1]
    grid = (pl.cdiv(vocab, _VBLK),)
    return pl.pallas_call(
        _mmT_body,
        grid=grid,
        in_specs=[
            pl.BlockSpec((D_MODEL, _VBLK), lambda j: (0, j)),
            pl.BlockSpec((BATCH, D_MODEL), lambda j: (0, 0)),
            pl.BlockSpec((1, _VBLK), lambda j: (0, j)),
        ],
        out_specs=pl.BlockSpec((_VBLK, BATCH), lambda j: (j, 0)),
        out_shape=jax.ShapeDtypeStruct((vocab, BATCH), jnp.float32),
    )(WT, emb, b2)


def kernel(x, table, W, b):
    table2 = _prep_tc(table.T)
    emb = _gather_sc(table2, x.astype(jnp.int32))
    logits_t = _project_tc_T(emb, W.T, b.reshape(1, -1))
    return logits_t.T
